# baseline (speedup 1.0000x reference)
import functools

import jax
import jax.numpy as jnp
from jax import lax
from jax.experimental import pallas as pl
from jax.experimental.pallas import tpu as pltpu
from jax.experimental.pallas import tpu_sc as plsc

NUM_CONDITIONS = 100000
D_MODEL = 64
D_PACK = 128
BATCH = 16384

_NUM_CORES = 2
_NUM_SUBCORES = 16
_NUM_WORKERS = _NUM_CORES * _NUM_SUBCORES
_B_PER_W = BATCH // _NUM_WORKERS
_LANES = 16


def _bcast_lane(vec, lane):
    idx = jnp.full((_LANES, 1), lane, jnp.int32)
    return lax.gather(
        vec, idx,
        lax.GatherDimensionNumbers(
            offset_dims=(), collapsed_slice_dims=(0,), start_index_map=(0,)),
        (1,),
        mode=lax.GatherScatterMode.PROMISE_IN_BOUNDS)


@functools.cache
def _build_gather():
    mesh = plsc.VectorSubcoreMesh(core_axis_name="c", subcore_axis_name="s")

    @functools.partial(
        pl.kernel,
        mesh=mesh,
        out_type=jax.ShapeDtypeStruct((BATCH, D_PACK), jnp.float32),
        compiler_params=pltpu.CompilerParams(use_tc_tiling_on_sc=False),
        scratch_types=[
            pltpu.VMEM((_B_PER_W,), jnp.int32),
            pltpu.VMEM((_B_PER_W,), jnp.int32),
            pltpu.VMEM((_B_PER_W, D_PACK), jnp.float32),
            pltpu.SemaphoreType.DMA,
        ],
    )
    def gather_kernel(table_hbm, idx_hbm, out_hbm, idx_v, pidx_v, rows_v,
                      sem):
        wid = lax.axis_index("s") * _NUM_CORES + lax.axis_index("c")
        base = wid * _B_PER_W
        pltpu.sync_copy(idx_hbm.at[pl.ds(base, _B_PER_W)], idx_v)

        def pidx_body(g, carry):
            v = idx_v[pl.ds(g * _LANES, _LANES)]
            pidx_v[pl.ds(g * _LANES, _LANES)] = jnp.right_shift(v, 1)
            return carry

        lax.fori_loop(0, _B_PER_W // _LANES, pidx_body, 0)
        pltpu.async_copy(table_hbm.at[pidx_v], rows_v, sem).wait()

        lanes = lax.iota(jnp.int32, _LANES)

        def sel_body(g, carry):
            idxg = idx_v[pl.ds(g * _LANES, _LANES)]
            odd = jnp.bitwise_and(idxg, 1).astype(jnp.float32)
            for l in range(_LANES):
                c = g * _LANES + l
                m = _bcast_lane(odd, l)
                for kk in range(D_MODEL // _LANES):
                    lo = rows_v[c, pl.ds(_LANES * kk, _LANES)]
                    hi = rows_v[c, pl.ds(D_MODEL + _LANES * kk, _LANES)]
                    rows_v[c, pl.ds(_LANES * kk, _LANES)] = lo + m * (hi - lo)
            return carry

        lax.fori_loop(0, _B_PER_W // _LANES, sel_body, 0)
        pltpu.sync_copy(rows_v, out_hbm.at[pl.ds(base, _B_PER_W)])

    return gather_kernel


def kernel(env_condition, table):
    idx = env_condition.astype(jnp.int32)
    tpack = table.reshape(NUM_CONDITIONS // 2, D_PACK)
    outp = _build_gather()(tpack, idx)
    return outp[:, :D_MODEL]


# speedup vs baseline: 1.1651x; 1.1651x over previous
import functools

import jax
import jax.numpy as jnp
from jax import lax
from jax.experimental import pallas as pl
from jax.experimental.pallas import tpu as pltpu
from jax.experimental.pallas import tpu_sc as plsc

NUM_CONDITIONS = 100000
D_MODEL = 64
D_PAD = 128
BATCH = 16384

_NUM_CORES = 2
_NUM_SUBCORES = 16
_NUM_WORKERS = _NUM_CORES * _NUM_SUBCORES
_B_PER_W = BATCH // _NUM_WORKERS


@functools.cache
def _build_gather():
    mesh = plsc.VectorSubcoreMesh(core_axis_name="c", subcore_axis_name="s")

    @functools.partial(
        pl.kernel,
        mesh=mesh,
        out_type=jax.ShapeDtypeStruct((BATCH, D_PAD), jnp.float32),
        compiler_params=pltpu.CompilerParams(use_tc_tiling_on_sc=False),
        scratch_types=[
            pltpu.VMEM((_B_PER_W,), jnp.int32),
            pltpu.VMEM((_B_PER_W, D_PAD), jnp.float32),
            pltpu.SemaphoreType.DMA,
        ],
    )
    def gather_kernel(table_hbm, idx_hbm, out_hbm, idx_v, rows_v, sem):
        wid = lax.axis_index("s") * _NUM_CORES + lax.axis_index("c")
        base = wid * _B_PER_W
        pltpu.sync_copy(idx_hbm.at[pl.ds(base, _B_PER_W)], idx_v)
        pltpu.async_copy(table_hbm.at[idx_v], rows_v, sem).wait()
        pltpu.sync_copy(rows_v, out_hbm.at[pl.ds(base, _B_PER_W)])

    return gather_kernel


def kernel(env_condition, table):
    idx = env_condition.astype(jnp.int32)
    tpad = jnp.pad(table, ((0, 0), (0, D_PAD - D_MODEL)))
    outp = _build_gather()(tpad, idx)
    return outp[:, :D_MODEL]


# speedup vs baseline: 1.3394x; 1.1496x over previous
import functools

import jax
import jax.numpy as jnp
from jax import lax
from jax.experimental import pallas as pl
from jax.experimental.pallas import tpu as pltpu
from jax.experimental.pallas import tpu_sc as plsc

NUM_CONDITIONS = 100000
D_MODEL = 64
BATCH = 16384

_NUM_CORES = 2
_NUM_SUBCORES = 16
_NUM_WORKERS = _NUM_CORES * _NUM_SUBCORES
_B_PER_W = BATCH // _NUM_WORKERS
_LANES = 16
_NGROUP = _B_PER_W // _LANES


@functools.cache
def _build_gather():
    mesh = plsc.VectorSubcoreMesh(core_axis_name="c", subcore_axis_name="s")

    @functools.partial(
        pl.kernel,
        mesh=mesh,
        out_type=jax.ShapeDtypeStruct((BATCH, D_MODEL), jnp.float32),
        compiler_params=pltpu.CompilerParams(use_tc_tiling_on_sc=True),
        scratch_types=[
            pltpu.VMEM((_B_PER_W,), jnp.int32),
            pltpu.VMEM((_B_PER_W, D_MODEL), jnp.float32),
            pltpu.SemaphoreType.DMA,
        ],
    )
    def gather_kernel(table_hbm, idx_hbm, out_hbm, idx_v, rows_v, sem):
        wid = lax.axis_index("s") * _NUM_CORES + lax.axis_index("c")
        base = wid * _B_PER_W
        pltpu.sync_copy(idx_hbm.at[pl.ds(base, _B_PER_W)], idx_v)

        def issue(g):
            vec = idx_v[pl.ds(g * _LANES, _LANES)]
            for l in range(_LANES):
                pltpu.async_copy(
                    table_hbm.at[vec[l]], rows_v.at[g * _LANES + l], sem)

        def drain():
            for _ in range(_LANES):
                pltpu.make_async_copy(
                    table_hbm.at[0], rows_v.at[0], sem).wait()

        issue(jnp.int32(0))

        def body(g, carry):
            issue(g)
            drain()
            return carry

        lax.fori_loop(1, _NGROUP, body, 0)
        drain()
        pltpu.sync_copy(rows_v, out_hbm.at[pl.ds(base, _B_PER_W)])

    return gather_kernel


def kernel(env_condition, table):
    idx = env_condition.astype(jnp.int32)
    return _build_gather()(table, idx)
